# baseline (speedup 1.0000x reference)
import functools

import jax
import jax.numpy as jnp
from jax.experimental import pallas as pl
from jax.experimental.pallas import tpu as pltpu

_NA = 3
_NC = 3
_G = 152
_GG = _G * _G
_SXY = 1.05
_OFF = 0.5 * (_SXY - 1.0)
_AW = (1.08, 3.42, 6.63)
_AH = (1.19, 4.41, 11.38)


def _decode_body(stride_ref, x_ref, o_ref):
    p = x_ref[0]
    stride = stride_ref[0]
    a = pl.program_id(1)

    aw = jnp.where(a == 0, _AW[0], jnp.where(a == 1, _AW[1], _AW[2]))
    ah = jnp.where(a == 0, _AH[0], jnp.where(a == 1, _AH[1], _AH[2]))
    anch = jnp.concatenate(
        [jnp.full((1, _G, _G), aw, jnp.float32),
         jnp.full((1, _G, _G), ah, jnp.float32)], axis=0)

    gx = jax.lax.broadcasted_iota(jnp.int32, (1, _G, _G), 2).astype(jnp.float32)
    gy = jax.lax.broadcasted_iota(jnp.int32, (1, _G, _G), 1).astype(jnp.float32)

    sig = jax.nn.sigmoid(p)
    ex = jnp.exp(p[2:4])

    d6, d7, d8, d9 = p[6], p[7], p[8], p[9]
    idx = jnp.where(d7 > d6, 1.0, 0.0)
    best = jnp.maximum(d6, d7)
    idx = jnp.where(d8 > best, 2.0, idx)
    best = jnp.maximum(best, d8)
    dirv = jnp.where(d9 > best, 3.0, idx)[None]

    val = jnp.concatenate([
        (sig[0:1] * _SXY - _OFF + gx) * stride,
        (sig[1:2] * _SXY - _OFF + gy) * stride,
        ex * anch,
        sig[4:6],
        dirv,
        sig[10:14],
    ], axis=0)

    for y in range(_G):
        o_ref[0, 0, pl.ds(y * _G, _G), :] = val[:, y, :].T


def _decode_half(stride, x, b0, nb):
    return pl.pallas_call(
        _decode_body,
        grid=(nb, _NA),
        in_specs=[
            pl.BlockSpec(memory_space=pltpu.SMEM),
            pl.BlockSpec((1, _NC + 11, _G, _G),
                         lambda i, j, b0=b0: (i + b0, j, 0, 0)),
        ],
        out_specs=pl.BlockSpec((1, 1, _GG, 11), lambda i, j: (i, j, 0, 0)),
        out_shape=jax.ShapeDtypeStruct((nb, _NA, _GG, 11), jnp.float32),
        compiler_params=pltpu.CompilerParams(
            dimension_semantics=("arbitrary", "arbitrary"),
        ),
    )(stride, x)


@functools.partial(jax.jit, static_argnums=())
def kernel(x, img_size):
    n = x.shape[0]
    stride = (jnp.float32(img_size) / _G).reshape(1)
    nb = n // 2
    r1 = _decode_half(stride, x, 0, nb)
    r2 = _decode_half(stride, x, nb, n - nb)
    return jnp.concatenate([r1, r2], axis=0).reshape(n, _NA * _GG, 11)


# speedup vs baseline: 1.9179x; 1.9179x over previous
import functools

import jax
import jax.numpy as jnp
from jax.experimental import pallas as pl
from jax.experimental.pallas import tpu as pltpu

_NA = 3
_NC = 3
_G = 152
_GG = _G * _G
_SXY = 1.05
_OFF = 0.5 * (_SXY - 1.0)
_AW = (1.08, 3.42, 6.63)
_AH = (1.19, 4.41, 11.38)
_R = 152
_YB = _G // _R


def _decode_body(stride_ref, x_ref, o_ref):
    p = x_ref[0]
    stride = stride_ref[0]
    j = pl.program_id(1)
    a = j // _YB
    y0 = (j - a * _YB) * _R

    aw = jnp.where(a == 0, _AW[0], jnp.where(a == 1, _AW[1], _AW[2]))
    ah = jnp.where(a == 0, _AH[0], jnp.where(a == 1, _AH[1], _AH[2]))
    anch = jnp.concatenate(
        [jnp.full((1, _R, _G), aw, jnp.float32),
         jnp.full((1, _R, _G), ah, jnp.float32)], axis=0)

    gx = jax.lax.broadcasted_iota(jnp.int32, (1, _R, _G), 2).astype(jnp.float32)
    gy = (jax.lax.broadcasted_iota(jnp.int32, (1, _R, _G), 1)
          + y0).astype(jnp.float32)

    sig = jax.nn.sigmoid(p)
    ex = jnp.exp(p[2:4])

    d6, d7, d8, d9 = p[6], p[7], p[8], p[9]
    idx = jnp.where(d7 > d6, 1.0, 0.0)
    best = jnp.maximum(d6, d7)
    idx = jnp.where(d8 > best, 2.0, idx)
    best = jnp.maximum(best, d8)
    dirv = jnp.where(d9 > best, 3.0, idx)[None]

    val = jnp.concatenate([
        (sig[0:1] * _SXY - _OFF + gx) * stride,
        (sig[1:2] * _SXY - _OFF + gy) * stride,
        ex * anch,
        sig[4:6],
        dirv,
        sig[10:14],
    ], axis=0)

    for y in range(_R):
        o_ref[0, 0, pl.ds(y * _G, _G), :] = val[:, y, :].T


@functools.partial(jax.jit, static_argnums=())
def kernel(x, img_size):
    n = x.shape[0]
    stride = (jnp.float32(img_size) / _G).reshape(1)

    return pl.pallas_call(
        _decode_body,
        grid=(n, _NA * _YB),
        in_specs=[
            pl.BlockSpec(memory_space=pltpu.SMEM),
            pl.BlockSpec((1, _NC + 11, _R, _G),
                         lambda i, j: (i, j // _YB, j % _YB, 0)),
        ],
        out_specs=pl.BlockSpec((1, 1, _R * _G, 11), lambda i, j: (i, j, 0, 0)),
        out_shape=jax.ShapeDtypeStruct((n, _NA, _GG, 11), jnp.float32),
        compiler_params=pltpu.CompilerParams(
            dimension_semantics=("arbitrary", "arbitrary"),
        ),
    )(stride, x).reshape(n, _NA * _GG, 11)
